# baseline (speedup 1.0000x reference)
import functools

import jax
import jax.numpy as jnp
from jax import lax
from jax.experimental import pallas as pl
from jax.experimental.pallas import tpu as pltpu
from jax.experimental.pallas import tpu_sc as plsc

_NUM_CORES = 2
_NUM_SUBCORES = 16
_NW = _NUM_CORES * _NUM_SUBCORES


@functools.partial(jax.jit, static_argnums=(2, 3, 4))
def _gather(idx, weight, N, D, K):
    n_per_w = N // _NW
    n_chunks = n_per_w // K
    mesh = plsc.VectorSubcoreMesh(
        core_axis_name="c",
        subcore_axis_name="s",
        num_cores=_NUM_CORES,
        num_subcores=_NUM_SUBCORES,
    )

    NBUF = 2

    @functools.partial(
        pl.kernel,
        out_type=jax.ShapeDtypeStruct((N, D), jnp.float32),
        mesh=mesh,
        scratch_types=[
            pltpu.VMEM((n_per_w,), jnp.int32),
            pltpu.VMEM((NBUF, K, D), jnp.float32),
            [pltpu.SemaphoreType.DMA] * NBUF,
            [pltpu.SemaphoreType.DMA] * NBUF,
            pltpu.SemaphoreType.DMA,
        ],
    )
    def k(idx_hbm, table_hbm, out_hbm, idx_v, rows_v, gsem, ssem, isem):
        T = idx_hbm.shape[1]
        wpr = T // n_per_w
        wid = lax.axis_index("s") * _NUM_CORES + lax.axis_index("c")
        base = wid * n_per_w
        row, col = wid // wpr, (wid % wpr) * n_per_w
        half = n_per_w // 2

        pltpu.sync_copy(
            idx_hbm.at[row, pl.ds(col, half)], idx_v.at[pl.ds(0, half)]
        )
        for b in range(NBUF - 1):
            pltpu.async_copy(
                table_hbm.at[idx_v.at[pl.ds(b * K, K)]], rows_v.at[b], gsem[b]
            )
        pltpu.async_copy(
            idx_hbm.at[row, pl.ds(col + half, half)],
            idx_v.at[pl.ds(half, half)],
            isem,
        )
        pltpu.make_async_copy(
            idx_hbm.at[row, pl.ds(col, half)],
            idx_v.at[pl.ds(half, half)],
            isem,
        ).wait()

        @pl.loop(0, n_chunks, step=NBUF)
        def _(j):
            for b in range(NBUF):
                cur = j + b
                bp = (b + NBUF - 1) % NBUF
                pltpu.make_async_copy(
                    table_hbm.at[idx_v.at[pl.ds(0, K)]], rows_v.at[b], gsem[b]
                ).wait()
                pltpu.async_copy(
                    rows_v.at[b], out_hbm.at[pl.ds(base + cur * K, K)], ssem[b]
                )
                pre = cur + NBUF - 1

                @pl.when(pre < n_chunks)
                def _():
                    @pl.when(cur >= 1)
                    def _():
                        pltpu.make_async_copy(
                            rows_v.at[bp], out_hbm.at[pl.ds(base, K)], ssem[bp]
                        ).wait()

                    pltpu.async_copy(
                        table_hbm.at[idx_v.at[pl.ds(pre * K, K)]],
                        rows_v.at[bp],
                        gsem[bp],
                    )

        for b in range(NBUF):
            pltpu.make_async_copy(
                rows_v.at[b], out_hbm.at[pl.ds(base, K)], ssem[b]
            ).wait()

    return k(idx, weight)


def kernel(input_ids, weight):
    B, T = input_ids.shape
    V, D = weight.shape
    N = B * T
    K = 16
    if input_ids.dtype != jnp.int32:
        input_ids = input_ids.astype(jnp.int32)
    out = _gather(input_ids, weight, N, D, K)
    return out.reshape(B, T, D)


# speedup vs baseline: 1.0536x; 1.0536x over previous
import functools

import jax
import jax.numpy as jnp
from jax import lax
from jax.experimental import pallas as pl
from jax.experimental.pallas import tpu as pltpu
from jax.experimental.pallas import tpu_sc as plsc

_NUM_CORES = 2
_NUM_SUBCORES = 16
_NW = _NUM_CORES * _NUM_SUBCORES


@functools.partial(jax.jit, static_argnums=(2, 3, 4))
def _gather(idx, weight, N, D, K):
    n_per_w = N // _NW
    n_chunks = n_per_w // K
    mesh = plsc.VectorSubcoreMesh(
        core_axis_name="c",
        subcore_axis_name="s",
        num_cores=_NUM_CORES,
        num_subcores=_NUM_SUBCORES,
    )

    NBUF = 4

    @functools.partial(
        pl.kernel,
        out_type=jax.ShapeDtypeStruct((N, D), jnp.float32),
        mesh=mesh,
        scratch_types=[
            pltpu.VMEM((n_per_w,), jnp.int32),
            pltpu.VMEM((NBUF, K, D), jnp.float32),
            [pltpu.SemaphoreType.DMA] * NBUF,
            [pltpu.SemaphoreType.DMA] * NBUF,
            pltpu.SemaphoreType.DMA,
        ],
    )
    def k(idx_hbm, table_hbm, out_hbm, idx_v, rows_v, gsem, ssem, isem):
        T = idx_hbm.shape[1]
        wpr = T // n_per_w
        wid = lax.axis_index("s") * _NUM_CORES + lax.axis_index("c")
        base = wid * n_per_w
        row, col = wid // wpr, (wid % wpr) * n_per_w
        half = n_per_w // 2

        pltpu.sync_copy(
            idx_hbm.at[row, pl.ds(col, half)], idx_v.at[pl.ds(0, half)]
        )
        for b in range(NBUF - 1):
            pltpu.async_copy(
                table_hbm.at[idx_v.at[pl.ds(b * K, K)]], rows_v.at[b], gsem[b]
            )
        pltpu.async_copy(
            idx_hbm.at[row, pl.ds(col + half, half)],
            idx_v.at[pl.ds(half, half)],
            isem,
        )
        pltpu.make_async_copy(
            idx_hbm.at[row, pl.ds(col, half)],
            idx_v.at[pl.ds(half, half)],
            isem,
        ).wait()

        @pl.loop(0, n_chunks, step=NBUF)
        def _(j):
            for b in range(NBUF):
                cur = j + b
                bp = (b + NBUF - 1) % NBUF
                pltpu.make_async_copy(
                    table_hbm.at[idx_v.at[pl.ds(0, K)]], rows_v.at[b], gsem[b]
                ).wait()
                pltpu.async_copy(
                    rows_v.at[b], out_hbm.at[pl.ds(base + cur * K, K)], ssem[b]
                )
                pre = cur + NBUF - 1

                @pl.when(pre < n_chunks)
                def _():
                    @pl.when(cur >= 1)
                    def _():
                        pltpu.make_async_copy(
                            rows_v.at[bp], out_hbm.at[pl.ds(base, K)], ssem[bp]
                        ).wait()

                    pltpu.async_copy(
                        table_hbm.at[idx_v.at[pl.ds(pre * K, K)]],
                        rows_v.at[bp],
                        gsem[bp],
                    )

        for b in range(NBUF):
            pltpu.make_async_copy(
                rows_v.at[b], out_hbm.at[pl.ds(base, K)], ssem[b]
            ).wait()

    return k(idx, weight)


def kernel(input_ids, weight):
    B, T = input_ids.shape
    V, D = weight.shape
    N = B * T
    K = 8
    if input_ids.dtype != jnp.int32:
        input_ids = input_ids.astype(jnp.int32)
    out = _gather(input_ids, weight, N, D, K)
    return out.reshape(B, T, D)
